# Initial kernel scaffold; baseline (speedup 1.0000x reference)
import jax
import jax.numpy as jnp
from jax.experimental import pallas as pl


def kernel(x, batch, W1, b1, W2, b2):
    raise NotImplementedError("write your pallas kernel here")


# speedup vs baseline: 2.9142x; 2.9142x over previous
import functools

import jax
import jax.numpy as jnp
from jax.experimental import pallas as pl
from jax.experimental.pallas import tpu as pltpu

_HIDDEN = 128
_NSEG = 1024
_R = 3200
_S = 64


def _fused_kernel(wlo_ref, whi_ref, x_ref, brow_ref, bcol_ref, w1_ref, b1_ref,
                  w2_ref, b2_ref, out_ref, sum_s, max_s, cnt_s):
    i = pl.program_id(0)
    nb = pl.num_programs(0)

    @pl.when(i == 0)
    def _init():
        sum_s[...] = jnp.zeros_like(sum_s)
        max_s[...] = jnp.zeros_like(max_s)
        cnt_s[...] = jnp.zeros_like(cnt_s)

    x = x_ref[...]
    h = jax.lax.dot_general(x, w1_ref[...], (((1,), (1,)), ((), ())),
                            preferred_element_type=jnp.float32)
    h = jnp.maximum(h + b1_ref[...], 0.0)
    att = jax.lax.dot_general(h, w2_ref[...], (((1,), (1,)), ((), ())),
                              preferred_element_type=jnp.float32)
    att = jax.nn.sigmoid(att + b2_ref[...])
    attended = h * att

    brow = brow_ref[0]
    bcol = bcol_ref[0]

    nxt = jnp.concatenate(
        [brow[:, 1:], jnp.full((1, 1), -1, jnp.int32)], axis=1)
    run_end = (brow != nxt).astype(jnp.float32)

    scanned = attended
    d = 1
    while d < _R:
        sh_v = jnp.concatenate(
            [jnp.zeros((d, _HIDDEN), jnp.float32), scanned[:-d, :]], axis=0)
        sh_i = jnp.concatenate(
            [jnp.full((d, 1), -1, jnp.int32), bcol[:-d, :]], axis=0)
        same = bcol == sh_i
        scanned = jnp.where(same, jnp.maximum(scanned, sh_v), scanned)
        d *= 2

    def _window(w, carry):
        base = w * _S
        iota_s = jax.lax.broadcasted_iota(jnp.int32, (_S, _R), 0)
        oh = ((brow - base) == iota_s).astype(jnp.float32)
        sums_u = jax.lax.dot_general(oh, attended, (((1,), (0,)), ((), ())),
                                     preferred_element_type=jnp.float32)
        cnts_u = jnp.sum(oh, axis=1, keepdims=True)
        sel = oh * run_end
        maxs_u = jax.lax.dot_general(sel, scanned, (((1,), (0,)), ((), ())),
                                     preferred_element_type=jnp.float32)
        sum_s[pl.ds(base, _S), :] += sums_u
        cnt_s[pl.ds(base, _S), :] += cnts_u
        max_s[pl.ds(base, _S), :] = jnp.maximum(max_s[pl.ds(base, _S), :],
                                                maxs_u)
        return carry

    jax.lax.fori_loop(wlo_ref[i], whi_ref[i] + 1, _window, 0)

    @pl.when(i == nb - 1)
    def _finish():
        cnt = cnt_s[...]
        out_ref[:, :_HIDDEN] = max_s[...]
        out_ref[:, _HIDDEN:] = sum_s[...] / jnp.maximum(cnt, 1.0)


@jax.jit
def kernel(x, batch, W1, b1, W2, b2):
    n = x.shape[0]
    assert n % _R == 0
    nb = n // _R
    batch = batch.astype(jnp.int32)
    brow = batch.reshape(nb, 1, _R)
    bcol = batch.reshape(nb, _R, 1)
    wlo = (batch[::_R] // _S).astype(jnp.int32)
    whi = (batch[_R - 1::_R] // _S).astype(jnp.int32)
    b1r = b1.reshape(1, _HIDDEN)
    b2r = b2.reshape(1, _HIDDEN)

    grid_spec = pltpu.PrefetchScalarGridSpec(
        num_scalar_prefetch=2,
        grid=(nb,),
        in_specs=[
            pl.BlockSpec((_R, _HIDDEN), lambda i, *_: (i, 0)),
            pl.BlockSpec((1, 1, _R), lambda i, *_: (i, 0, 0)),
            pl.BlockSpec((1, _R, 1), lambda i, *_: (i, 0, 0)),
            pl.BlockSpec((_HIDDEN, _HIDDEN), lambda i, *_: (0, 0)),
            pl.BlockSpec((1, _HIDDEN), lambda i, *_: (0, 0)),
            pl.BlockSpec((_HIDDEN, _HIDDEN), lambda i, *_: (0, 0)),
            pl.BlockSpec((1, _HIDDEN), lambda i, *_: (0, 0)),
        ],
        out_specs=pl.BlockSpec((_NSEG, 2 * _HIDDEN), lambda i, *_: (0, 0)),
        scratch_shapes=[
            pltpu.VMEM((_NSEG, _HIDDEN), jnp.float32),
            pltpu.VMEM((_NSEG, _HIDDEN), jnp.float32),
            pltpu.VMEM((_NSEG, 1), jnp.float32),
        ],
    )
    out = pl.pallas_call(
        _fused_kernel,
        grid_spec=grid_spec,
        out_shape=jax.ShapeDtypeStruct((_NSEG, 2 * _HIDDEN), jnp.float32),
        compiler_params=pltpu.CompilerParams(
            dimension_semantics=("arbitrary",)),
    )(wlo, whi, x, brow, bcol, W1, b1r, W2, b2r)
    return out
